# baseline (speedup 1.0000x reference)
import functools

import jax
import jax.numpy as jnp
from jax import lax
from jax.experimental import pallas as pl
from jax.experimental.pallas import tpu as pltpu
from jax.experimental.pallas import tpu_sc as plsc

E = 64
K = 2
D = 1024
F = 768
T = 4096
TB = 512
NTB = T // TB
B = 128
NBLK = T * K // B + E
P = NBLK * B
NW = 32
TPW = T // NW
SUB = 16
NSUB = TPW // SUB


def _router_body(x_ref, gw_ref, e0_ref, e1_ref, w0_ref, w1_ref,
                 r0_ref, r1_ref, h0_ref, h1_ref):
    x = x_ref[...]
    gw = gw_ref[...]
    logits = lax.dot_general(x, gw, (((1,), (1,)), ((), ())),
                             preferred_element_type=jnp.float32)
    iota_e = lax.broadcasted_iota(jnp.int32, (TB, E), 1)
    m1 = jnp.max(logits, axis=-1, keepdims=True)
    i1 = jnp.min(jnp.where(logits == m1, iota_e, E), axis=-1)
    l2 = jnp.where(iota_e == i1[:, None], -jnp.inf, logits)
    m2 = jnp.max(l2, axis=-1, keepdims=True)
    i2 = jnp.min(jnp.where(l2 == m2, iota_e, E), axis=-1)
    t = jnp.exp(m2[:, 0] - m1[:, 0])
    p0 = 1.0 / (1.0 + t)
    p1 = 1.0 - p0

    oh0 = (iota_e == i1[:, None]).astype(jnp.float32)
    oh1 = (iota_e == i2[:, None]).astype(jnp.float32)
    r_i = lax.broadcasted_iota(jnp.int32, (TB, TB), 0)
    c_i = lax.broadcasted_iota(jnp.int32, (TB, TB), 1)
    ltri = (c_i < r_i).astype(jnp.float32)
    cum0 = lax.dot_general(ltri, oh0, (((1,), (0,)), ((), ())),
                           preferred_element_type=jnp.float32)
    cum1 = lax.dot_general(ltri, oh1, (((1,), (0,)), ((), ())),
                           preferred_element_type=jnp.float32)
    r0 = jnp.sum(cum0 * oh0, axis=-1).astype(jnp.int32)
    r1 = jnp.sum(cum1 * oh1, axis=-1).astype(jnp.int32)

    e0_ref[...] = i1[None, None, :]
    e1_ref[...] = i2[None, None, :]
    w0_ref[...] = p0[None, None, :]
    w1_ref[...] = p1[None, None, :]
    r0_ref[...] = r0[None, None, :]
    r1_ref[...] = r1[None, None, :]
    h0_ref[...] = jnp.sum(oh0, axis=0).astype(jnp.int32)[None, None, :]
    h1_ref[...] = jnp.sum(oh1, axis=0).astype(jnp.int32)[None, None, :]


def _run_router(x, gate_weight, interpret=False):
    tok3 = lambda dt: jax.ShapeDtypeStruct((NTB, 1, TB), dt)
    hist3 = jax.ShapeDtypeStruct((NTB, 1, E), jnp.int32)
    tok_spec = pl.BlockSpec((1, 1, TB), lambda i: (i, 0, 0))
    hist_spec = pl.BlockSpec((1, 1, E), lambda i: (i, 0, 0))
    return pl.pallas_call(
        _router_body,
        grid=(NTB,),
        in_specs=[pl.BlockSpec((TB, D), lambda i: (i, 0)),
                  pl.BlockSpec((E, D), lambda i: (0, 0))],
        out_specs=[tok_spec, tok_spec, tok_spec, tok_spec, tok_spec, tok_spec,
                   hist_spec, hist_spec],
        out_shape=[tok3(jnp.int32), tok3(jnp.int32),
                   tok3(jnp.float32), tok3(jnp.float32),
                   tok3(jnp.int32), tok3(jnp.int32), hist3, hist3],
        interpret=interpret,
    )(x, gate_weight)


def _routing_tables(h0, h1):
    c0 = jnp.sum(h0, axis=0)
    c1 = jnp.sum(h1, axis=0)
    counts = c0 + c1
    nblk_e = (counts + B - 1) // B
    ends = jnp.cumsum(nblk_e)
    blk_start = ends - nblk_e
    pstart = B * blk_start
    total_blk = ends[E - 1]
    bids = jnp.arange(NBLK, dtype=jnp.int32)
    be = jnp.sum((bids[:, None] >= ends[None, :]).astype(jnp.int32), axis=1)
    block_expert = jnp.where(bids < total_blk, be, -1).astype(jnp.int32)
    cb0 = jnp.cumsum(h0, axis=0) - h0
    cb1 = jnp.cumsum(h1, axis=0) - h1
    base0 = (pstart[None, :] + cb0).astype(jnp.int32)
    base1 = (pstart[None, :] + c0[None, :] + cb1).astype(jnp.int32)
    return block_expert, base0, base1


def _k2_body(x_hbm, e0_hbm, e1_hbm, r0_hbm, r1_hbm, b0_hbm, b1_hbm,
             xs_hbm, d0_hbm, d1_hbm,
             ev, rv, dv, basev, xrows, sem):
    wid = lax.axis_index("s") * 2 + lax.axis_index("c")
    row0 = wid * NSUB
    for e_hbm, r_hbm, b_hbm, d_hbm in (
            (e0_hbm, r0_hbm, b0_hbm, d0_hbm),
            (e1_hbm, r1_hbm, b1_hbm, d1_hbm)):
        pltpu.sync_copy(e_hbm.at[pl.ds(row0, NSUB)], ev)
        pltpu.sync_copy(r_hbm.at[pl.ds(row0, NSUB)], rv)
        pltpu.sync_copy(b_hbm, basev)
        for sub in range(NSUB):
            e16 = ev[sub, :]
            r16 = rv[sub, :]
            tstart = wid * TPW + sub * SUB
            tb = tstart // TB
            flat = tb * E + e16
            base = plsc.load_gather(basev, [flat >> 4, flat & 15])
            dv[sub, :] = base + r16
        pltpu.sync_copy(dv, d_hbm.at[pl.ds(row0, NSUB)])
        for sub in range(NSUB):
            tstart = wid * TPW + sub * SUB
            pltpu.sync_copy(x_hbm.at[pl.ds(tstart, SUB)], xrows)
            dest = dv[sub, :]
            pltpu.async_copy(xrows, xs_hbm.at[dest], sem).wait()


def _run_k2(x, e0, e1, r0, r1, base0, base1):
    kern = functools.partial(
        pl.kernel,
        mesh=plsc.VectorSubcoreMesh(core_axis_name="c", subcore_axis_name="s"),
        compiler_params=pltpu.CompilerParams(needs_layout_passes=False),
        out_type=[jax.ShapeDtypeStruct((P, D), jnp.float32),
                  jax.ShapeDtypeStruct((T // SUB, SUB), jnp.int32),
                  jax.ShapeDtypeStruct((T // SUB, SUB), jnp.int32)],
        scratch_types=[pltpu.VMEM((NSUB, SUB), jnp.int32),
                       pltpu.VMEM((NSUB, SUB), jnp.int32),
                       pltpu.VMEM((NSUB, SUB), jnp.int32),
                       pltpu.VMEM((NTB * E // 16, 16), jnp.int32),
                       pltpu.VMEM((SUB, D), jnp.float32),
                       pltpu.SemaphoreType.DMA],
    )(_k2_body)
    return kern(x, e0, e1, r0, r1, base0, base1)


def _k3_body(be_ref, xs_ref, w1_ref, w3_ref, w2_ref, ys_ref):
    b = pl.program_id(0)
    e = be_ref[b]

    @pl.when(e >= 0)
    def _():
        xv = xs_ref[...].astype(jnp.bfloat16)
        a = jnp.dot(xv, w1_ref[0], preferred_element_type=jnp.float32)
        g = jnp.dot(xv, w3_ref[0], preferred_element_type=jnp.float32)
        h = (a * jax.nn.sigmoid(a) * g).astype(jnp.bfloat16)
        ys_ref[...] = jnp.dot(h, w2_ref[0], preferred_element_type=jnp.float32)


def _run_k3(block_expert, xs, w1, w3, w2, interpret=False):
    def wmap(i, be):
        return (jnp.maximum(be[i], 0), 0, 0)
    grid_spec = pltpu.PrefetchScalarGridSpec(
        num_scalar_prefetch=1,
        grid=(NBLK,),
        in_specs=[pl.BlockSpec((B, D), lambda i, be: (i, 0)),
                  pl.BlockSpec((1, D, F), wmap),
                  pl.BlockSpec((1, D, F), wmap),
                  pl.BlockSpec((1, F, D), wmap)],
        out_specs=pl.BlockSpec((B, D), lambda i, be: (i, 0)),
    )
    return pl.pallas_call(
        _k3_body,
        grid_spec=grid_spec,
        out_shape=jax.ShapeDtypeStruct((P, D), jnp.float32),
        interpret=interpret,
    )(block_expert, xs, w1, w3, w2)


def _k4_body(ys_hbm, d0_hbm, d1_hbm, w0_hbm, w1_hbm, out_hbm,
             dv0, dv1, wv0, wv1, rows0, rows1, obuf, sem):
    wid = lax.axis_index("s") * 2 + lax.axis_index("c")
    row0 = wid * NSUB
    pltpu.sync_copy(d0_hbm.at[pl.ds(row0, NSUB)], dv0)
    pltpu.sync_copy(d1_hbm.at[pl.ds(row0, NSUB)], dv1)
    pltpu.sync_copy(w0_hbm.at[pl.ds(row0, NSUB)], wv0)
    pltpu.sync_copy(w1_hbm.at[pl.ds(row0, NSUB)], wv1)
    for sub in range(NSUB):
        idx0 = dv0[sub, :]
        idx1 = dv1[sub, :]
        pltpu.async_copy(ys_hbm.at[idx0], rows0, sem).wait()
        pltpu.async_copy(ys_hbm.at[idx1], rows1, sem).wait()
        w0v = wv0[sub, :]
        w1v = wv1[sub, :]
        for i in range(SUB):
            s0 = w0v[i]
            s1 = w1v[i]

            def body(j, _):
                sl = pl.ds(j * 16, 16)
                obuf[i, sl] = s0 * rows0[i, sl] + s1 * rows1[i, sl]
                return 0

            lax.fori_loop(0, D // 16, body, 0)
        tstart = wid * TPW + sub * SUB
        pltpu.sync_copy(obuf, out_hbm.at[pl.ds(tstart, SUB)])


def _run_k4(ys, d0, d1, w0, w1):
    kern = functools.partial(
        pl.kernel,
        mesh=plsc.VectorSubcoreMesh(core_axis_name="c", subcore_axis_name="s"),
        compiler_params=pltpu.CompilerParams(needs_layout_passes=False),
        out_type=jax.ShapeDtypeStruct((T, D), jnp.float32),
        scratch_types=[pltpu.VMEM((NSUB, SUB), jnp.int32),
                       pltpu.VMEM((NSUB, SUB), jnp.int32),
                       pltpu.VMEM((NSUB, SUB), jnp.float32),
                       pltpu.VMEM((NSUB, SUB), jnp.float32),
                       pltpu.VMEM((SUB, D), jnp.float32),
                       pltpu.VMEM((SUB, D), jnp.float32),
                       pltpu.VMEM((SUB, D), jnp.float32),
                       pltpu.SemaphoreType.DMA],
    )(_k4_body)
    return kern(ys, d0, d1, w0, w1)


def kernel(hidden_states, gate_weight, w1, w3, w2):
    x = hidden_states
    e0, e1, w0, w1t, r0, r1, h0, h1 = _run_router(x, gate_weight)
    block_expert, base0, base1 = _routing_tables(h0[:, 0, :], h1[:, 0, :])
    to16 = lambda a: a.reshape(T // SUB, SUB)
    xs, d0, d1 = _run_k2(x, to16(e0), to16(e1), to16(r0), to16(r1),
                         base0.reshape(NTB * E // 16, 16),
                         base1.reshape(NTB * E // 16, 16))
    ys = _run_k3(block_expert, xs, w1.astype(jnp.bfloat16),
                 w3.astype(jnp.bfloat16), w2.astype(jnp.bfloat16))
    out = _run_k4(ys, d0, d1, to16(w0), to16(w1t))
    return out


# speedup vs baseline: 1.4830x; 1.4830x over previous
import functools

import jax
import jax.numpy as jnp
from jax import lax
from jax.experimental import pallas as pl
from jax.experimental.pallas import tpu as pltpu
from jax.experimental.pallas import tpu_sc as plsc

E = 64
K = 2
D = 1024
F = 768
T = 4096
TB = 512
NTB = T // TB
B = 128
NBLK = T * K // B + E
P = NBLK * B
NW = 32
TPW = T // NW
SUB = 16
NSUB = TPW // SUB


def _router_body(x_ref, gw_ref, e0_ref, e1_ref, w0_ref, w1_ref,
                 r0_ref, r1_ref, h0_ref, h1_ref):
    x = x_ref[...]
    gw = gw_ref[...]
    logits = lax.dot_general(x, gw, (((1,), (1,)), ((), ())),
                             preferred_element_type=jnp.float32)
    iota_e = lax.broadcasted_iota(jnp.int32, (TB, E), 1)
    m1 = jnp.max(logits, axis=-1, keepdims=True)
    i1 = jnp.min(jnp.where(logits == m1, iota_e, E), axis=-1)
    l2 = jnp.where(iota_e == i1[:, None], -jnp.inf, logits)
    m2 = jnp.max(l2, axis=-1, keepdims=True)
    i2 = jnp.min(jnp.where(l2 == m2, iota_e, E), axis=-1)
    t = jnp.exp(m2[:, 0] - m1[:, 0])
    p0 = 1.0 / (1.0 + t)
    p1 = 1.0 - p0

    oh0 = (iota_e == i1[:, None]).astype(jnp.float32)
    oh1 = (iota_e == i2[:, None]).astype(jnp.float32)
    r_i = lax.broadcasted_iota(jnp.int32, (TB, TB), 0)
    c_i = lax.broadcasted_iota(jnp.int32, (TB, TB), 1)
    ltri = (c_i < r_i).astype(jnp.float32)
    cum0 = lax.dot_general(ltri, oh0, (((1,), (0,)), ((), ())),
                           preferred_element_type=jnp.float32)
    cum1 = lax.dot_general(ltri, oh1, (((1,), (0,)), ((), ())),
                           preferred_element_type=jnp.float32)
    r0 = jnp.sum(cum0 * oh0, axis=-1).astype(jnp.int32)
    r1 = jnp.sum(cum1 * oh1, axis=-1).astype(jnp.int32)

    e0_ref[...] = i1[None, None, :]
    e1_ref[...] = i2[None, None, :]
    w0_ref[...] = p0[None, None, :]
    w1_ref[...] = p1[None, None, :]
    r0_ref[...] = r0[None, None, :]
    r1_ref[...] = r1[None, None, :]
    h0_ref[...] = jnp.sum(oh0, axis=0).astype(jnp.int32)[None, None, :]
    h1_ref[...] = jnp.sum(oh1, axis=0).astype(jnp.int32)[None, None, :]


def _run_router(x, gate_weight, interpret=False):
    tok3 = lambda dt: jax.ShapeDtypeStruct((NTB, 1, TB), dt)
    hist3 = jax.ShapeDtypeStruct((NTB, 1, E), jnp.int32)
    tok_spec = pl.BlockSpec((1, 1, TB), lambda i: (i, 0, 0))
    hist_spec = pl.BlockSpec((1, 1, E), lambda i: (i, 0, 0))
    return pl.pallas_call(
        _router_body,
        grid=(NTB,),
        in_specs=[pl.BlockSpec((TB, D), lambda i: (i, 0)),
                  pl.BlockSpec((E, D), lambda i: (0, 0))],
        out_specs=[tok_spec, tok_spec, tok_spec, tok_spec, tok_spec, tok_spec,
                   hist_spec, hist_spec],
        out_shape=[tok3(jnp.int32), tok3(jnp.int32),
                   tok3(jnp.float32), tok3(jnp.float32),
                   tok3(jnp.int32), tok3(jnp.int32), hist3, hist3],
        interpret=interpret,
    )(x, gate_weight)


def _routing_tables(h0, h1):
    c0 = jnp.sum(h0, axis=0)
    c1 = jnp.sum(h1, axis=0)
    counts = c0 + c1
    nblk_e = (counts + B - 1) // B
    ends = jnp.cumsum(nblk_e)
    blk_start = ends - nblk_e
    pstart = B * blk_start
    total_blk = ends[E - 1]
    bids = jnp.arange(NBLK, dtype=jnp.int32)
    be = jnp.sum((bids[:, None] >= ends[None, :]).astype(jnp.int32), axis=1)
    block_expert = jnp.where(bids < total_blk, be, -1).astype(jnp.int32)
    cb0 = jnp.cumsum(h0, axis=0) - h0
    cb1 = jnp.cumsum(h1, axis=0) - h1
    base0 = (pstart[None, :] + cb0).astype(jnp.int32)
    base1 = (pstart[None, :] + c0[None, :] + cb1).astype(jnp.int32)
    return block_expert, base0, base1


def _k2_body(x_hbm, e0_hbm, e1_hbm, r0_hbm, r1_hbm, b0_hbm, b1_hbm,
             xs_hbm, d0_hbm, d1_hbm,
             ev, rv, dv0, dv1, basev, xrows, sem0, sem1):
    wid = lax.axis_index("s") * 2 + lax.axis_index("c")
    row0 = wid * NSUB
    for e_hbm, r_hbm, b_hbm, d_hbm, dv in (
            (e0_hbm, r0_hbm, b0_hbm, d0_hbm, dv0),
            (e1_hbm, r1_hbm, b1_hbm, d1_hbm, dv1)):
        pltpu.sync_copy(e_hbm.at[pl.ds(row0, NSUB)], ev)
        pltpu.sync_copy(r_hbm.at[pl.ds(row0, NSUB)], rv)
        pltpu.sync_copy(b_hbm, basev)
        for sub in range(NSUB):
            e16 = ev[sub, :]
            r16 = rv[sub, :]
            tstart = wid * TPW + sub * SUB
            tb = tstart // TB
            flat = tb * E + e16
            base = plsc.load_gather(basev, [flat >> 4, flat & 15])
            dv[sub, :] = base + r16
        pltpu.sync_copy(dv, d_hbm.at[pl.ds(row0, NSUB)])
    sems = (sem0, sem1)
    pend = [[], []]
    for sub in range(NSUB):
        slot = sub % 2
        for h in pend[slot]:
            h.wait()
        pend[slot] = []
        tstart = wid * TPW + sub * SUB
        pltpu.sync_copy(x_hbm.at[pl.ds(tstart, SUB)], xrows.at[slot])
        pend[slot].append(
            pltpu.async_copy(xrows.at[slot], xs_hbm.at[dv0[sub, :]], sems[slot]))
        pend[slot].append(
            pltpu.async_copy(xrows.at[slot], xs_hbm.at[dv1[sub, :]], sems[slot]))
    for slot in (0, 1):
        for h in pend[slot]:
            h.wait()


def _run_k2(x, e0, e1, r0, r1, base0, base1):
    kern = functools.partial(
        pl.kernel,
        mesh=plsc.VectorSubcoreMesh(core_axis_name="c", subcore_axis_name="s"),
        compiler_params=pltpu.CompilerParams(needs_layout_passes=False),
        out_type=[jax.ShapeDtypeStruct((P, D), jnp.float32),
                  jax.ShapeDtypeStruct((T // SUB, SUB), jnp.int32),
                  jax.ShapeDtypeStruct((T // SUB, SUB), jnp.int32)],
        scratch_types=[pltpu.VMEM((NSUB, SUB), jnp.int32),
                       pltpu.VMEM((NSUB, SUB), jnp.int32),
                       pltpu.VMEM((NSUB, SUB), jnp.int32),
                       pltpu.VMEM((NSUB, SUB), jnp.int32),
                       pltpu.VMEM((NTB * E // 16, 16), jnp.int32),
                       pltpu.VMEM((2, SUB, D), jnp.float32),
                       pltpu.SemaphoreType.DMA,
                       pltpu.SemaphoreType.DMA],
    )(_k2_body)
    return kern(x, e0, e1, r0, r1, base0, base1)


def _k3_body(be_ref, xs_ref, w1_ref, w3_ref, w2_ref, ys_ref):
    b = pl.program_id(0)
    e = be_ref[b]

    @pl.when(e >= 0)
    def _():
        xv = xs_ref[...]
        a = jnp.dot(xv, w1_ref[0], preferred_element_type=jnp.float32)
        g = jnp.dot(xv, w3_ref[0], preferred_element_type=jnp.float32)
        h = a * jax.nn.sigmoid(a) * g
        ys_ref[...] = jnp.dot(h, w2_ref[0], preferred_element_type=jnp.float32)


def _run_k3(block_expert, xs, w1, w3, w2, interpret=False):
    def wmap(i, be):
        return (jnp.maximum(be[i], 0), 0, 0)
    grid_spec = pltpu.PrefetchScalarGridSpec(
        num_scalar_prefetch=1,
        grid=(NBLK,),
        in_specs=[pl.BlockSpec((B, D),
                               lambda i, be: (jnp.where(be[i] >= 0, i, 0), 0)),
                  pl.BlockSpec((1, D, F), wmap),
                  pl.BlockSpec((1, D, F), wmap),
                  pl.BlockSpec((1, F, D), wmap)],
        out_specs=pl.BlockSpec((B, D), lambda i, be: (i, 0)),
    )
    return pl.pallas_call(
        _k3_body,
        grid_spec=grid_spec,
        out_shape=jax.ShapeDtypeStruct((P, D), jnp.float32),
        interpret=interpret,
    )(block_expert, xs, w1, w3, w2)


def _k4_body(ys_hbm, d0_hbm, d1_hbm, w0_hbm, w1_hbm, out_hbm,
             dv0, dv1, wv0, wv1, rows0, rows1, obuf, sem0, sem1):
    wid = lax.axis_index("s") * 2 + lax.axis_index("c")
    row0 = wid * NSUB
    pltpu.sync_copy(d0_hbm.at[pl.ds(row0, NSUB)], dv0)
    pltpu.sync_copy(d1_hbm.at[pl.ds(row0, NSUB)], dv1)
    pltpu.sync_copy(w0_hbm.at[pl.ds(row0, NSUB)], wv0)
    pltpu.sync_copy(w1_hbm.at[pl.ds(row0, NSUB)], wv1)
    sems = (sem0, sem1)

    def gather(sub):
        slot = sub % 2
        return [pltpu.async_copy(ys_hbm.at[dv0[sub, :]], rows0.at[slot],
                                 sems[slot]),
                pltpu.async_copy(ys_hbm.at[dv1[sub, :]], rows1.at[slot],
                                 sems[slot])]

    pend = gather(0)
    for sub in range(NSUB):
        slot = sub % 2
        for h in pend:
            h.wait()
        if sub + 1 < NSUB:
            pend = gather(sub + 1)
        w0v = wv0[sub, :]
        w1v = wv1[sub, :]
        for i in range(SUB):
            s0 = w0v[i]
            s1 = w1v[i]

            def body(j, _):
                for u in range(4):
                    sl = pl.ds(j * 64 + u * 16, 16)
                    obuf[i, sl] = (s0 * rows0[slot, i, sl] +
                                   s1 * rows1[slot, i, sl])
                return 0

            lax.fori_loop(0, D // 64, body, 0)
        tstart = wid * TPW + sub * SUB
        pltpu.sync_copy(obuf, out_hbm.at[pl.ds(tstart, SUB)])


def _run_k4(ys, d0, d1, w0, w1):
    kern = functools.partial(
        pl.kernel,
        mesh=plsc.VectorSubcoreMesh(core_axis_name="c", subcore_axis_name="s"),
        compiler_params=pltpu.CompilerParams(needs_layout_passes=False),
        out_type=jax.ShapeDtypeStruct((T, D), jnp.float32),
        scratch_types=[pltpu.VMEM((NSUB, SUB), jnp.int32),
                       pltpu.VMEM((NSUB, SUB), jnp.int32),
                       pltpu.VMEM((NSUB, SUB), jnp.float32),
                       pltpu.VMEM((NSUB, SUB), jnp.float32),
                       pltpu.VMEM((2, SUB, D), jnp.float32),
                       pltpu.VMEM((2, SUB, D), jnp.float32),
                       pltpu.VMEM((SUB, D), jnp.float32),
                       pltpu.SemaphoreType.DMA,
                       pltpu.SemaphoreType.DMA],
    )(_k4_body)
    return kern(ys, d0, d1, w0, w1)


def kernel(hidden_states, gate_weight, w1, w3, w2):
    x = hidden_states
    e0, e1, w0, w1t, r0, r1, h0, h1 = _run_router(x, gate_weight)
    block_expert, base0, base1 = _routing_tables(h0[:, 0, :], h1[:, 0, :])
    to16 = lambda a: a.reshape(T // SUB, SUB)
    xs, d0, d1 = _run_k2(x, to16(e0), to16(e1), to16(r0), to16(r1),
                         base0.reshape(NTB * E // 16, 16),
                         base1.reshape(NTB * E // 16, 16))
    ys = _run_k3(block_expert, xs, w1, w3, w2)
    out = _run_k4(ys, d0, d1, to16(w0), to16(w1t))
    return out


# speedup vs baseline: 1.4846x; 1.0011x over previous
import functools

import jax
import jax.numpy as jnp
from jax import lax
from jax.experimental import pallas as pl
from jax.experimental.pallas import tpu as pltpu
from jax.experimental.pallas import tpu_sc as plsc

E = 64
K = 2
D = 1024
F = 768
T = 4096
TB = 512
NTB = T // TB
B = 128
NBLK = T * K // B + E
P = NBLK * B
NW = 32
TPW = T // NW
SUB = 16
NSUB = TPW // SUB


def _router_body(x_ref, gw_ref, e0_ref, e1_ref, w0_ref, w1_ref,
                 r0_ref, r1_ref, h0_ref, h1_ref):
    x = x_ref[...]
    gw = gw_ref[...]
    logits = lax.dot_general(x, gw, (((1,), (1,)), ((), ())),
                             preferred_element_type=jnp.float32)
    iota_e = lax.broadcasted_iota(jnp.int32, (TB, E), 1)
    m1 = jnp.max(logits, axis=-1, keepdims=True)
    i1 = jnp.min(jnp.where(logits == m1, iota_e, E), axis=-1)
    l2 = jnp.where(iota_e == i1[:, None], -jnp.inf, logits)
    m2 = jnp.max(l2, axis=-1, keepdims=True)
    i2 = jnp.min(jnp.where(l2 == m2, iota_e, E), axis=-1)
    t = jnp.exp(m2[:, 0] - m1[:, 0])
    p0 = 1.0 / (1.0 + t)
    p1 = 1.0 - p0

    oh0 = (iota_e == i1[:, None]).astype(jnp.float32)
    oh1 = (iota_e == i2[:, None]).astype(jnp.float32)
    r_i = lax.broadcasted_iota(jnp.int32, (TB, TB), 0)
    c_i = lax.broadcasted_iota(jnp.int32, (TB, TB), 1)
    ltri = (c_i < r_i).astype(jnp.float32)
    cum0 = lax.dot_general(ltri, oh0, (((1,), (0,)), ((), ())),
                           preferred_element_type=jnp.float32)
    cum1 = lax.dot_general(ltri, oh1, (((1,), (0,)), ((), ())),
                           preferred_element_type=jnp.float32)
    r0 = jnp.sum(cum0 * oh0, axis=-1).astype(jnp.int32)
    r1 = jnp.sum(cum1 * oh1, axis=-1).astype(jnp.int32)

    e0_ref[...] = i1[None, None, :]
    e1_ref[...] = i2[None, None, :]
    w0_ref[...] = p0[None, None, :]
    w1_ref[...] = p1[None, None, :]
    r0_ref[...] = r0[None, None, :]
    r1_ref[...] = r1[None, None, :]
    h0_ref[...] = jnp.sum(oh0, axis=0).astype(jnp.int32)[None, None, :]
    h1_ref[...] = jnp.sum(oh1, axis=0).astype(jnp.int32)[None, None, :]


def _run_router(x, gate_weight, interpret=False):
    tok3 = lambda dt: jax.ShapeDtypeStruct((NTB, 1, TB), dt)
    hist3 = jax.ShapeDtypeStruct((NTB, 1, E), jnp.int32)
    tok_spec = pl.BlockSpec((1, 1, TB), lambda i: (i, 0, 0))
    hist_spec = pl.BlockSpec((1, 1, E), lambda i: (i, 0, 0))
    return pl.pallas_call(
        _router_body,
        grid=(NTB,),
        in_specs=[pl.BlockSpec((TB, D), lambda i: (i, 0)),
                  pl.BlockSpec((E, D), lambda i: (0, 0))],
        out_specs=[tok_spec, tok_spec, tok_spec, tok_spec, tok_spec, tok_spec,
                   hist_spec, hist_spec],
        out_shape=[tok3(jnp.int32), tok3(jnp.int32),
                   tok3(jnp.float32), tok3(jnp.float32),
                   tok3(jnp.int32), tok3(jnp.int32), hist3, hist3],
        interpret=interpret,
    )(x, gate_weight)


def _routing_tables(h0, h1):
    c0 = jnp.sum(h0, axis=0)
    c1 = jnp.sum(h1, axis=0)
    counts = c0 + c1
    nblk_e = (counts + B - 1) // B
    ends = jnp.cumsum(nblk_e)
    blk_start = ends - nblk_e
    pstart = B * blk_start
    total_blk = ends[E - 1]
    bids = jnp.arange(NBLK, dtype=jnp.int32)
    be = jnp.sum((bids[:, None] >= ends[None, :]).astype(jnp.int32), axis=1)
    block_expert = jnp.where(bids < total_blk, be, -1).astype(jnp.int32)
    cb0 = jnp.cumsum(h0, axis=0) - h0
    cb1 = jnp.cumsum(h1, axis=0) - h1
    base0 = (pstart[None, :] + cb0).astype(jnp.int32)
    base1 = (pstart[None, :] + c0[None, :] + cb1).astype(jnp.int32)
    return block_expert, base0, base1


def _k2_body(x_hbm, e0_hbm, e1_hbm, r0_hbm, r1_hbm, b0_hbm, b1_hbm,
             xs_hbm, d0_hbm, d1_hbm,
             ev, rv, dv0, dv1, basev, xrows, sem0, sem1):
    wid = lax.axis_index("s") * 2 + lax.axis_index("c")
    row0 = wid * NSUB
    for e_hbm, r_hbm, b_hbm, d_hbm, dv in (
            (e0_hbm, r0_hbm, b0_hbm, d0_hbm, dv0),
            (e1_hbm, r1_hbm, b1_hbm, d1_hbm, dv1)):
        pltpu.sync_copy(e_hbm.at[pl.ds(row0, NSUB)], ev)
        pltpu.sync_copy(r_hbm.at[pl.ds(row0, NSUB)], rv)
        pltpu.sync_copy(b_hbm, basev)
        for sub in range(NSUB):
            e16 = ev[sub, :]
            r16 = rv[sub, :]
            tstart = wid * TPW + sub * SUB
            tb = tstart // TB
            flat = tb * E + e16
            base = plsc.load_gather(basev, [flat >> 4, flat & 15])
            dv[sub, :] = base + r16
        pltpu.sync_copy(dv, d_hbm.at[pl.ds(row0, NSUB)])
    sems = (sem0, sem1)
    pend = [[], []]
    for sub in range(NSUB):
        slot = sub % 2
        for h in pend[slot]:
            h.wait()
        pend[slot] = []
        tstart = wid * TPW + sub * SUB
        pltpu.sync_copy(x_hbm.at[pl.ds(tstart, SUB)], xrows.at[slot])
        pend[slot].append(
            pltpu.async_copy(xrows.at[slot], xs_hbm.at[dv0[sub, :]], sems[slot]))
        pend[slot].append(
            pltpu.async_copy(xrows.at[slot], xs_hbm.at[dv1[sub, :]], sems[slot]))
    for slot in (0, 1):
        for h in pend[slot]:
            h.wait()


def _run_k2(x, e0, e1, r0, r1, base0, base1):
    kern = functools.partial(
        pl.kernel,
        mesh=plsc.VectorSubcoreMesh(core_axis_name="c", subcore_axis_name="s"),
        compiler_params=pltpu.CompilerParams(needs_layout_passes=False),
        out_type=[jax.ShapeDtypeStruct((P, D), jnp.float32),
                  jax.ShapeDtypeStruct((T // SUB, SUB), jnp.int32),
                  jax.ShapeDtypeStruct((T // SUB, SUB), jnp.int32)],
        scratch_types=[pltpu.VMEM((NSUB, SUB), jnp.int32),
                       pltpu.VMEM((NSUB, SUB), jnp.int32),
                       pltpu.VMEM((NSUB, SUB), jnp.int32),
                       pltpu.VMEM((NSUB, SUB), jnp.int32),
                       pltpu.VMEM((NTB * E // 16, 16), jnp.int32),
                       pltpu.VMEM((2, SUB, D), jnp.float32),
                       pltpu.SemaphoreType.DMA,
                       pltpu.SemaphoreType.DMA],
    )(_k2_body)
    return kern(x, e0, e1, r0, r1, base0, base1)


def _k3_body(be_ref, xs_ref, w1_ref, w3_ref, w2_ref, ys_ref):
    b = pl.program_id(0)
    e = be_ref[b]

    @pl.when(e >= 0)
    def _():
        xv = xs_ref[...]
        a = jnp.dot(xv, w1_ref[0], preferred_element_type=jnp.float32,
                    precision=lax.Precision.DEFAULT)
        g = jnp.dot(xv, w3_ref[0], preferred_element_type=jnp.float32,
                    precision=lax.Precision.DEFAULT)
        h = a * jax.nn.sigmoid(a) * g
        ys_ref[...] = jnp.dot(h, w2_ref[0], preferred_element_type=jnp.float32,
                              precision=lax.Precision.DEFAULT)


def _run_k3(block_expert, xs, w1, w3, w2, interpret=False):
    def wmap(i, be):
        return (jnp.maximum(be[i], 0), 0, 0)
    grid_spec = pltpu.PrefetchScalarGridSpec(
        num_scalar_prefetch=1,
        grid=(NBLK,),
        in_specs=[pl.BlockSpec((B, D),
                               lambda i, be: (jnp.where(be[i] >= 0, i, 0), 0)),
                  pl.BlockSpec((1, D, F), wmap),
                  pl.BlockSpec((1, D, F), wmap),
                  pl.BlockSpec((1, F, D), wmap)],
        out_specs=pl.BlockSpec((B, D), lambda i, be: (i, 0)),
    )
    return pl.pallas_call(
        _k3_body,
        grid_spec=grid_spec,
        out_shape=jax.ShapeDtypeStruct((P, D), jnp.float32),
        interpret=interpret,
    )(block_expert, xs, w1, w3, w2)


def _k4_body(ys_hbm, d0_hbm, d1_hbm, w0_hbm, w1_hbm, out_hbm,
             dv0, dv1, wv0, wv1, rows0, rows1, obuf, sem0, sem1):
    wid = lax.axis_index("s") * 2 + lax.axis_index("c")
    row0 = wid * NSUB
    pltpu.sync_copy(d0_hbm.at[pl.ds(row0, NSUB)], dv0)
    pltpu.sync_copy(d1_hbm.at[pl.ds(row0, NSUB)], dv1)
    pltpu.sync_copy(w0_hbm.at[pl.ds(row0, NSUB)], wv0)
    pltpu.sync_copy(w1_hbm.at[pl.ds(row0, NSUB)], wv1)
    sems = (sem0, sem1)

    def gather(sub):
        slot = sub % 2
        return [pltpu.async_copy(ys_hbm.at[dv0[sub, :]], rows0.at[slot],
                                 sems[slot]),
                pltpu.async_copy(ys_hbm.at[dv1[sub, :]], rows1.at[slot],
                                 sems[slot])]

    pend = gather(0)
    for sub in range(NSUB):
        slot = sub % 2
        for h in pend:
            h.wait()
        if sub + 1 < NSUB:
            pend = gather(sub + 1)
        w0v = wv0[sub, :]
        w1v = wv1[sub, :]
        for i in range(SUB):
            s0 = w0v[i]
            s1 = w1v[i]

            def body(j, _):
                for u in range(4):
                    sl = pl.ds(j * 64 + u * 16, 16)
                    obuf[i, sl] = (s0 * rows0[slot, i, sl] +
                                   s1 * rows1[slot, i, sl])
                return 0

            lax.fori_loop(0, D // 64, body, 0)
        tstart = wid * TPW + sub * SUB
        pltpu.sync_copy(obuf, out_hbm.at[pl.ds(tstart, SUB)])


def _run_k4(ys, d0, d1, w0, w1):
    kern = functools.partial(
        pl.kernel,
        mesh=plsc.VectorSubcoreMesh(core_axis_name="c", subcore_axis_name="s"),
        compiler_params=pltpu.CompilerParams(needs_layout_passes=False),
        out_type=jax.ShapeDtypeStruct((T, D), jnp.float32),
        scratch_types=[pltpu.VMEM((NSUB, SUB), jnp.int32),
                       pltpu.VMEM((NSUB, SUB), jnp.int32),
                       pltpu.VMEM((NSUB, SUB), jnp.float32),
                       pltpu.VMEM((NSUB, SUB), jnp.float32),
                       pltpu.VMEM((2, SUB, D), jnp.float32),
                       pltpu.VMEM((2, SUB, D), jnp.float32),
                       pltpu.VMEM((SUB, D), jnp.float32),
                       pltpu.SemaphoreType.DMA,
                       pltpu.SemaphoreType.DMA],
    )(_k4_body)
    return kern(ys, d0, d1, w0, w1)


def kernel(hidden_states, gate_weight, w1, w3, w2):
    x = hidden_states
    e0, e1, w0, w1t, r0, r1, h0, h1 = _run_router(x, gate_weight)
    block_expert, base0, base1 = _routing_tables(h0[:, 0, :], h1[:, 0, :])
    to16 = lambda a: a.reshape(T // SUB, SUB)
    xs, d0, d1 = _run_k2(x, to16(e0), to16(e1), to16(r0), to16(r1),
                         base0.reshape(NTB * E // 16, 16),
                         base1.reshape(NTB * E // 16, 16))
    ys = _run_k3(block_expert, xs, w1, w3, w2)
    out = _run_k4(ys, d0, d1, to16(w0), to16(w1t))
    return out


# speedup vs baseline: 1.4976x; 1.0088x over previous
import functools

import jax
import jax.numpy as jnp
from jax import lax
from jax.experimental import pallas as pl
from jax.experimental.pallas import tpu as pltpu
from jax.experimental.pallas import tpu_sc as plsc

E = 64
K = 2
D = 1024
F = 768
T = 4096
TB = 512
NTB = T // TB
B = 128
NBLK = T * K // B + E
P = NBLK * B
NW = 32
TPW = T // NW
SUB = 16
NSUB = TPW // SUB


def _router_body(x_ref, gw_ref, e0_ref, e1_ref, w0_ref, w1_ref,
                 r0_ref, r1_ref, h0_ref, h1_ref):
    x = x_ref[...]
    gw = gw_ref[...]
    logits = lax.dot_general(x, gw, (((1,), (1,)), ((), ())),
                             preferred_element_type=jnp.float32)
    iota_e = lax.broadcasted_iota(jnp.int32, (TB, E), 1)
    m1 = jnp.max(logits, axis=-1, keepdims=True)
    i1 = jnp.min(jnp.where(logits == m1, iota_e, E), axis=-1)
    l2 = jnp.where(iota_e == i1[:, None], -jnp.inf, logits)
    m2 = jnp.max(l2, axis=-1, keepdims=True)
    i2 = jnp.min(jnp.where(l2 == m2, iota_e, E), axis=-1)
    t = jnp.exp(m2[:, 0] - m1[:, 0])
    p0 = 1.0 / (1.0 + t)
    p1 = 1.0 - p0

    oh0 = (iota_e == i1[:, None]).astype(jnp.float32)
    oh1 = (iota_e == i2[:, None]).astype(jnp.float32)
    r_i = lax.broadcasted_iota(jnp.int32, (TB, TB), 0)
    c_i = lax.broadcasted_iota(jnp.int32, (TB, TB), 1)
    ltri = (c_i < r_i).astype(jnp.float32)
    cum0 = lax.dot_general(ltri, oh0, (((1,), (0,)), ((), ())),
                           preferred_element_type=jnp.float32)
    cum1 = lax.dot_general(ltri, oh1, (((1,), (0,)), ((), ())),
                           preferred_element_type=jnp.float32)
    r0 = jnp.sum(cum0 * oh0, axis=-1).astype(jnp.int32)
    r1 = jnp.sum(cum1 * oh1, axis=-1).astype(jnp.int32)

    e0_ref[...] = i1[None, None, :]
    e1_ref[...] = i2[None, None, :]
    w0_ref[...] = p0[None, None, :]
    w1_ref[...] = p1[None, None, :]
    r0_ref[...] = r0[None, None, :]
    r1_ref[...] = r1[None, None, :]
    h0_ref[...] = jnp.sum(oh0, axis=0).astype(jnp.int32)[None, None, :]
    h1_ref[...] = jnp.sum(oh1, axis=0).astype(jnp.int32)[None, None, :]


def _run_router(x, gate_weight, interpret=False):
    tok3 = lambda dt: jax.ShapeDtypeStruct((NTB, 1, TB), dt)
    hist3 = jax.ShapeDtypeStruct((NTB, 1, E), jnp.int32)
    tok_spec = pl.BlockSpec((1, 1, TB), lambda i: (i, 0, 0))
    hist_spec = pl.BlockSpec((1, 1, E), lambda i: (i, 0, 0))
    return pl.pallas_call(
        _router_body,
        grid=(NTB,),
        in_specs=[pl.BlockSpec((TB, D), lambda i: (i, 0)),
                  pl.BlockSpec((E, D), lambda i: (0, 0))],
        out_specs=[tok_spec, tok_spec, tok_spec, tok_spec, tok_spec, tok_spec,
                   hist_spec, hist_spec],
        out_shape=[tok3(jnp.int32), tok3(jnp.int32),
                   tok3(jnp.float32), tok3(jnp.float32),
                   tok3(jnp.int32), tok3(jnp.int32), hist3, hist3],
        interpret=interpret,
    )(x, gate_weight)


def _routing_tables(h0, h1):
    c0 = jnp.sum(h0, axis=0)
    c1 = jnp.sum(h1, axis=0)
    counts = c0 + c1
    nblk_e = (counts + B - 1) // B
    ends = jnp.cumsum(nblk_e)
    blk_start = ends - nblk_e
    pstart = B * blk_start
    total_blk = ends[E - 1]
    bids = jnp.arange(NBLK, dtype=jnp.int32)
    be = jnp.sum((bids[:, None] >= ends[None, :]).astype(jnp.int32), axis=1)
    block_expert = jnp.where(bids < total_blk, be, -1).astype(jnp.int32)
    cb0 = jnp.cumsum(h0, axis=0) - h0
    cb1 = jnp.cumsum(h1, axis=0) - h1
    base0 = (pstart[None, :] + cb0).astype(jnp.int32)
    base1 = (pstart[None, :] + c0[None, :] + cb1).astype(jnp.int32)
    return block_expert, base0, base1


def _k2_body(x_hbm, e0_hbm, e1_hbm, r0_hbm, r1_hbm, b0_hbm, b1_hbm,
             w0_hbm, w1_hbm,
             xs_hbm, d0_hbm, d1_hbm, ws_hbm,
             ev, rv, dv0, dv1, wv, basev, xrows, sem0, sem1, semw):
    wid = lax.axis_index("s") * 2 + lax.axis_index("c")
    row0 = wid * NSUB
    wpend = []
    for ci, (e_hbm, r_hbm, b_hbm, d_hbm, w_hbm, dv) in enumerate((
            (e0_hbm, r0_hbm, b0_hbm, d0_hbm, w0_hbm, dv0),
            (e1_hbm, r1_hbm, b1_hbm, d1_hbm, w1_hbm, dv1))):
        pltpu.sync_copy(e_hbm.at[pl.ds(row0, NSUB)], ev)
        pltpu.sync_copy(r_hbm.at[pl.ds(row0, NSUB)], rv)
        pltpu.sync_copy(w_hbm.at[pl.ds(row0, NSUB)], wv.at[ci])
        pltpu.sync_copy(b_hbm, basev)
        for sub in range(NSUB):
            e16 = ev[sub, :]
            r16 = rv[sub, :]
            tstart = wid * TPW + sub * SUB
            tb = tstart // TB
            flat = tb * E + e16
            base = plsc.load_gather(basev, [flat >> 4, flat & 15])
            dv[sub, :] = base + r16
        pltpu.sync_copy(dv, d_hbm.at[pl.ds(row0, NSUB)])
        for sub in range(NSUB):
            wpend.append(
                pltpu.async_copy(wv.at[ci, sub], ws_hbm.at[dv[sub, :]], semw))
    sems = (sem0, sem1)
    pend = [[], []]
    for sub in range(NSUB):
        slot = sub % 2
        for h in pend[slot]:
            h.wait()
        pend[slot] = []
        tstart = wid * TPW + sub * SUB
        pltpu.sync_copy(x_hbm.at[pl.ds(tstart, SUB)], xrows.at[slot])
        pend[slot].append(
            pltpu.async_copy(xrows.at[slot], xs_hbm.at[dv0[sub, :]], sems[slot]))
        pend[slot].append(
            pltpu.async_copy(xrows.at[slot], xs_hbm.at[dv1[sub, :]], sems[slot]))
    for slot in (0, 1):
        for h in pend[slot]:
            h.wait()
    for h in wpend:
        h.wait()


def _run_k2(x, e0, e1, r0, r1, base0, base1, w0, w1):
    kern = functools.partial(
        pl.kernel,
        mesh=plsc.VectorSubcoreMesh(core_axis_name="c", subcore_axis_name="s"),
        compiler_params=pltpu.CompilerParams(needs_layout_passes=False),
        out_type=[jax.ShapeDtypeStruct((P, D), jnp.float32),
                  jax.ShapeDtypeStruct((T // SUB, SUB), jnp.int32),
                  jax.ShapeDtypeStruct((T // SUB, SUB), jnp.int32),
                  jax.ShapeDtypeStruct((P,), jnp.float32)],
        scratch_types=[pltpu.VMEM((NSUB, SUB), jnp.int32),
                       pltpu.VMEM((NSUB, SUB), jnp.int32),
                       pltpu.VMEM((NSUB, SUB), jnp.int32),
                       pltpu.VMEM((NSUB, SUB), jnp.int32),
                       pltpu.VMEM((2, NSUB, SUB), jnp.float32),
                       pltpu.VMEM((NTB * E // 16, 16), jnp.int32),
                       pltpu.VMEM((2, SUB, D), jnp.float32),
                       pltpu.SemaphoreType.DMA,
                       pltpu.SemaphoreType.DMA,
                       pltpu.SemaphoreType.DMA],
    )(_k2_body)
    return kern(x, e0, e1, r0, r1, base0, base1, w0, w1)


def _k3_body(be_ref, xs_ref, ws_ref, w1_ref, w3_ref, w2_ref, ys_ref):
    b = pl.program_id(0)
    e = be_ref[b]

    @pl.when(e >= 0)
    def _():
        xv = xs_ref[...]
        a = jnp.dot(xv, w1_ref[0], preferred_element_type=jnp.float32)
        g = jnp.dot(xv, w3_ref[0], preferred_element_type=jnp.float32)
        h = a * jax.nn.sigmoid(a) * g
        y = jnp.dot(h, w2_ref[0], preferred_element_type=jnp.float32)
        ys_ref[...] = y * ws_ref[0, 0, :][:, None]


def _run_k3(block_expert, xs, ws, w1, w3, w2, interpret=False):
    def wmap(i, be):
        return (jnp.maximum(be[i], 0), 0, 0)

    def amap(i, be):
        return (jnp.where(be[i] >= 0, i, 0), 0)
    grid_spec = pltpu.PrefetchScalarGridSpec(
        num_scalar_prefetch=1,
        grid=(NBLK,),
        in_specs=[pl.BlockSpec((B, D), amap),
                  pl.BlockSpec((1, 1, B),
                               lambda i, be: (jnp.where(be[i] >= 0, i, 0), 0, 0)),
                  pl.BlockSpec((1, D, F), wmap),
                  pl.BlockSpec((1, D, F), wmap),
                  pl.BlockSpec((1, F, D), wmap)],
        out_specs=pl.BlockSpec((B, D), lambda i, be: (i, 0)),
    )
    return pl.pallas_call(
        _k3_body,
        grid_spec=grid_spec,
        out_shape=jax.ShapeDtypeStruct((P, D), jnp.float32),
        interpret=interpret,
    )(block_expert, xs, ws.reshape(NBLK, 1, B), w1, w3, w2)


def _k4_body(ys_hbm, d0_hbm, d1_hbm, out_hbm,
             dv0, dv1, rows1, obuf, sem0, sem1):
    wid = lax.axis_index("s") * 2 + lax.axis_index("c")
    row0 = wid * NSUB
    pltpu.sync_copy(d0_hbm.at[pl.ds(row0, NSUB)], dv0)
    pltpu.sync_copy(d1_hbm.at[pl.ds(row0, NSUB)], dv1)
    sems = (sem0, sem1)

    def gather(sub):
        slot = sub % 2
        return [pltpu.async_copy(ys_hbm.at[dv0[sub, :]], obuf.at[slot],
                                 sems[slot]),
                pltpu.async_copy(ys_hbm.at[dv1[sub, :]], rows1.at[slot],
                                 sems[slot])]

    pend = gather(0)
    for sub in range(NSUB):
        slot = sub % 2
        for h in pend:
            h.wait()
        if sub + 1 < NSUB:
            pend = gather(sub + 1)
        for i in range(SUB):
            def body(j, _):
                for u in range(4):
                    sl = pl.ds(j * 64 + u * 16, 16)
                    plsc.addupdate(obuf.at[slot, i, sl], rows1[slot, i, sl])
                return 0

            lax.fori_loop(0, D // 64, body, 0)
        tstart = wid * TPW + sub * SUB
        pltpu.sync_copy(obuf.at[slot], out_hbm.at[pl.ds(tstart, SUB)])


def _run_k4(ys, d0, d1):
    kern = functools.partial(
        pl.kernel,
        mesh=plsc.VectorSubcoreMesh(core_axis_name="c", subcore_axis_name="s"),
        compiler_params=pltpu.CompilerParams(needs_layout_passes=False),
        out_type=jax.ShapeDtypeStruct((T, D), jnp.float32),
        scratch_types=[pltpu.VMEM((NSUB, SUB), jnp.int32),
                       pltpu.VMEM((NSUB, SUB), jnp.int32),
                       pltpu.VMEM((2, SUB, D), jnp.float32),
                       pltpu.VMEM((2, SUB, D), jnp.float32),
                       pltpu.SemaphoreType.DMA,
                       pltpu.SemaphoreType.DMA],
    )(_k4_body)
    return kern(ys, d0, d1)


def kernel(hidden_states, gate_weight, w1, w3, w2):
    x = hidden_states
    e0, e1, w0, w1t, r0, r1, h0, h1 = _run_router(x, gate_weight)
    block_expert, base0, base1 = _routing_tables(h0[:, 0, :], h1[:, 0, :])
    to16 = lambda a: a.reshape(T // SUB, SUB)
    xs, d0, d1, ws = _run_k2(x, to16(e0), to16(e1), to16(r0), to16(r1),
                             base0.reshape(NTB * E // 16, 16),
                             base1.reshape(NTB * E // 16, 16),
                             to16(w0), to16(w1t))
    ys = _run_k3(block_expert, xs, ws, w1, w3, w2)
    out = _run_k4(ys, d0, d1)
    return out


# speedup vs baseline: 1.5570x; 1.0397x over previous
import functools

import jax
import jax.numpy as jnp
from jax import lax
from jax.experimental import pallas as pl
from jax.experimental.pallas import tpu as pltpu
from jax.experimental.pallas import tpu_sc as plsc

E = 64
K = 2
D = 1024
F = 768
T = 4096
TB = 512
NTB = T // TB
B = 128
NBLK = T * K // B + E
P = NBLK * B
NW = 32
TPW = T // NW
SUB = 16
NSUB = TPW // SUB


def _router_body(x_ref, gw_ref, e0_ref, e1_ref, w0_ref, w1_ref,
                 r0_ref, r1_ref, h0_ref, h1_ref):
    x = x_ref[...]
    gw = gw_ref[...]
    logits = lax.dot_general(x, gw, (((1,), (1,)), ((), ())),
                             preferred_element_type=jnp.float32)
    iota_f = lax.broadcasted_iota(jnp.int32, (TB, E), 1).astype(jnp.float32)
    m1 = jnp.max(logits, axis=-1, keepdims=True)
    i1 = jnp.min(jnp.where(logits == m1, iota_f, float(E)), axis=-1)
    l2 = jnp.where(iota_f == i1[:, None], -jnp.inf, logits)
    m2 = jnp.max(l2, axis=-1, keepdims=True)
    i2 = jnp.min(jnp.where(l2 == m2, iota_f, float(E)), axis=-1)
    p0 = jax.nn.sigmoid(m1[:, 0] - m2[:, 0])
    p1 = 1.0 - p0

    oh0 = (iota_f == i1[:, None]).astype(jnp.float32)
    oh1 = (iota_f == i2[:, None]).astype(jnp.float32)
    r_i = lax.broadcasted_iota(jnp.int32, (TB, TB), 0)
    c_i = lax.broadcasted_iota(jnp.int32, (TB, TB), 1)
    ltri = (c_i < r_i).astype(jnp.float32)
    cum0 = lax.dot_general(ltri, oh0, (((1,), (0,)), ((), ())),
                           preferred_element_type=jnp.float32)
    cum1 = lax.dot_general(ltri, oh1, (((1,), (0,)), ((), ())),
                           preferred_element_type=jnp.float32)
    ones_e = jnp.ones((E, 1), jnp.float32)
    r0 = lax.dot_general(cum0 * oh0, ones_e, (((1,), (0,)), ((), ())),
                         preferred_element_type=jnp.float32)[:, 0]
    r1 = lax.dot_general(cum1 * oh1, ones_e, (((1,), (0,)), ((), ())),
                         preferred_element_type=jnp.float32)[:, 0]
    r0 = r0.astype(jnp.int32)
    r1 = r1.astype(jnp.int32)

    e0_ref[...] = i1.astype(jnp.int32)[None, None, :]
    e1_ref[...] = i2.astype(jnp.int32)[None, None, :]
    w0_ref[...] = p0[None, None, :]
    w1_ref[...] = p1[None, None, :]
    r0_ref[...] = r0[None, None, :]
    r1_ref[...] = r1[None, None, :]
    h0_ref[...] = jnp.sum(oh0, axis=0).astype(jnp.int32)[None, None, :]
    h1_ref[...] = jnp.sum(oh1, axis=0).astype(jnp.int32)[None, None, :]


def _run_router(x, gate_weight, interpret=False):
    tok3 = lambda dt: jax.ShapeDtypeStruct((NTB, 1, TB), dt)
    hist3 = jax.ShapeDtypeStruct((NTB, 1, E), jnp.int32)
    tok_spec = pl.BlockSpec((1, 1, TB), lambda i: (i, 0, 0))
    hist_spec = pl.BlockSpec((1, 1, E), lambda i: (i, 0, 0))
    return pl.pallas_call(
        _router_body,
        grid=(NTB,),
        in_specs=[pl.BlockSpec((TB, D), lambda i: (i, 0)),
                  pl.BlockSpec((E, D), lambda i: (0, 0))],
        out_specs=[tok_spec, tok_spec, tok_spec, tok_spec, tok_spec, tok_spec,
                   hist_spec, hist_spec],
        out_shape=[tok3(jnp.int32), tok3(jnp.int32),
                   tok3(jnp.float32), tok3(jnp.float32),
                   tok3(jnp.int32), tok3(jnp.int32), hist3, hist3],
        interpret=interpret,
    )(x, gate_weight)


def _routing_tables(h0, h1):
    c0 = jnp.sum(h0, axis=0)
    c1 = jnp.sum(h1, axis=0)
    counts = c0 + c1
    nblk_e = (counts + B - 1) // B
    ends = jnp.cumsum(nblk_e)
    blk_start = ends - nblk_e
    pstart = B * blk_start
    total_blk = ends[E - 1]
    bids = jnp.arange(NBLK, dtype=jnp.int32)
    be = jnp.sum((bids[:, None] >= ends[None, :]).astype(jnp.int32), axis=1)
    block_expert = jnp.where(bids < total_blk, be, -1).astype(jnp.int32)
    cb0 = jnp.cumsum(h0, axis=0) - h0
    cb1 = jnp.cumsum(h1, axis=0) - h1
    base0 = (pstart[None, :] + cb0).astype(jnp.int32)
    base1 = (pstart[None, :] + c0[None, :] + cb1).astype(jnp.int32)
    return block_expert, base0, base1


def _k2_body(x_hbm, e0_hbm, e1_hbm, r0_hbm, r1_hbm, b0_hbm, b1_hbm,
             w0_hbm, w1_hbm,
             xs_hbm, d0_hbm, d1_hbm, ws_hbm,
             ev, rv, dv0, dv1, wv, basev, xrows, sem0, sem1, semw):
    wid = lax.axis_index("s") * 2 + lax.axis_index("c")
    row0 = wid * NSUB
    wpend = []
    for ci, (e_hbm, r_hbm, b_hbm, d_hbm, w_hbm, dv) in enumerate((
            (e0_hbm, r0_hbm, b0_hbm, d0_hbm, w0_hbm, dv0),
            (e1_hbm, r1_hbm, b1_hbm, d1_hbm, w1_hbm, dv1))):
        pltpu.sync_copy(e_hbm.at[pl.ds(row0, NSUB)], ev)
        pltpu.sync_copy(r_hbm.at[pl.ds(row0, NSUB)], rv)
        pltpu.sync_copy(w_hbm.at[pl.ds(row0, NSUB)], wv.at[ci])
        pltpu.sync_copy(b_hbm, basev)
        for sub in range(NSUB):
            e16 = ev[sub, :]
            r16 = rv[sub, :]
            tstart = wid * TPW + sub * SUB
            tb = tstart // TB
            flat = tb * E + e16
            base = plsc.load_gather(basev, [flat >> 4, flat & 15])
            dv[sub, :] = base + r16
        pltpu.sync_copy(dv, d_hbm.at[pl.ds(row0, NSUB)])
        for sub in range(NSUB):
            wpend.append(
                pltpu.async_copy(wv.at[ci, sub], ws_hbm.at[dv[sub, :]], semw))
    sems = (sem0, sem1)
    pend = [[], []]
    for sub in range(NSUB):
        slot = sub % 2
        for h in pend[slot]:
            h.wait()
        pend[slot] = []
        tstart = wid * TPW + sub * SUB
        pltpu.sync_copy(x_hbm.at[pl.ds(tstart, SUB)], xrows.at[slot])
        pend[slot].append(
            pltpu.async_copy(xrows.at[slot], xs_hbm.at[dv0[sub, :]], sems[slot]))
        pend[slot].append(
            pltpu.async_copy(xrows.at[slot], xs_hbm.at[dv1[sub, :]], sems[slot]))
    for slot in (0, 1):
        for h in pend[slot]:
            h.wait()
    for h in wpend:
        h.wait()


def _run_k2(x, e0, e1, r0, r1, base0, base1, w0, w1):
    kern = functools.partial(
        pl.kernel,
        mesh=plsc.VectorSubcoreMesh(core_axis_name="c", subcore_axis_name="s"),
        compiler_params=pltpu.CompilerParams(needs_layout_passes=False),
        out_type=[jax.ShapeDtypeStruct((P, D), jnp.float32),
                  jax.ShapeDtypeStruct((T // SUB, SUB), jnp.int32),
                  jax.ShapeDtypeStruct((T // SUB, SUB), jnp.int32),
                  jax.ShapeDtypeStruct((P,), jnp.float32)],
        scratch_types=[pltpu.VMEM((NSUB, SUB), jnp.int32),
                       pltpu.VMEM((NSUB, SUB), jnp.int32),
                       pltpu.VMEM((NSUB, SUB), jnp.int32),
                       pltpu.VMEM((NSUB, SUB), jnp.int32),
                       pltpu.VMEM((2, NSUB, SUB), jnp.float32),
                       pltpu.VMEM((NTB * E // 16, 16), jnp.int32),
                       pltpu.VMEM((2, SUB, D), jnp.float32),
                       pltpu.SemaphoreType.DMA,
                       pltpu.SemaphoreType.DMA,
                       pltpu.SemaphoreType.DMA],
    )(_k2_body)
    return kern(x, e0, e1, r0, r1, base0, base1, w0, w1)


def _k3_body(be_ref, xs_ref, ws_ref, w1_ref, w3_ref, w2_ref, ys_ref):
    b = pl.program_id(0)
    e = be_ref[b]

    @pl.when(e >= 0)
    def _():
        xv = xs_ref[...]
        a = jnp.dot(xv, w1_ref[0], preferred_element_type=jnp.float32)
        g = jnp.dot(xv, w3_ref[0], preferred_element_type=jnp.float32)
        h = a * jax.nn.sigmoid(a) * g
        y = jnp.dot(h, w2_ref[0], preferred_element_type=jnp.float32)
        ys_ref[...] = y * ws_ref[0, 0, :][:, None]


def _run_k3(block_expert, xs, ws, w1, w3, w2, interpret=False):
    def wmap(i, be):
        return (jnp.maximum(be[i], 0), 0, 0)

    def amap(i, be):
        return (jnp.where(be[i] >= 0, i, 0), 0)
    grid_spec = pltpu.PrefetchScalarGridSpec(
        num_scalar_prefetch=1,
        grid=(NBLK,),
        in_specs=[pl.BlockSpec((B, D), amap),
                  pl.BlockSpec((1, 1, B),
                               lambda i, be: (jnp.where(be[i] >= 0, i, 0), 0, 0)),
                  pl.BlockSpec((1, D, F), wmap),
                  pl.BlockSpec((1, D, F), wmap),
                  pl.BlockSpec((1, F, D), wmap)],
        out_specs=pl.BlockSpec((B, D), lambda i, be: (i, 0)),
    )
    return pl.pallas_call(
        _k3_body,
        grid_spec=grid_spec,
        out_shape=jax.ShapeDtypeStruct((P, D), jnp.float32),
        interpret=interpret,
    )(block_expert, xs, ws.reshape(NBLK, 1, B), w1, w3, w2)


def _k4_body(ys_hbm, d0_hbm, d1_hbm, out_hbm,
             dv0, dv1, rows1, obuf, sem0, sem1):
    wid = lax.axis_index("s") * 2 + lax.axis_index("c")
    row0 = wid * NSUB
    pltpu.sync_copy(d0_hbm.at[pl.ds(row0, NSUB)], dv0)
    pltpu.sync_copy(d1_hbm.at[pl.ds(row0, NSUB)], dv1)
    sems = (sem0, sem1)

    def gather(sub):
        slot = sub % 2
        return [pltpu.async_copy(ys_hbm.at[dv0[sub, :]], obuf.at[slot],
                                 sems[slot]),
                pltpu.async_copy(ys_hbm.at[dv1[sub, :]], rows1.at[slot],
                                 sems[slot])]

    pend = gather(0)
    for sub in range(NSUB):
        slot = sub % 2
        for h in pend:
            h.wait()
        if sub + 1 < NSUB:
            pend = gather(sub + 1)
        for i in range(SUB):
            def body(j, _):
                for u in range(4):
                    sl = pl.ds(j * 64 + u * 16, 16)
                    plsc.addupdate(obuf.at[slot, i, sl], rows1[slot, i, sl])
                return 0

            lax.fori_loop(0, D // 64, body, 0)
        tstart = wid * TPW + sub * SUB
        pltpu.sync_copy(obuf.at[slot], out_hbm.at[pl.ds(tstart, SUB)])


def _run_k4(ys, d0, d1):
    kern = functools.partial(
        pl.kernel,
        mesh=plsc.VectorSubcoreMesh(core_axis_name="c", subcore_axis_name="s"),
        compiler_params=pltpu.CompilerParams(needs_layout_passes=False),
        out_type=jax.ShapeDtypeStruct((T, D), jnp.float32),
        scratch_types=[pltpu.VMEM((NSUB, SUB), jnp.int32),
                       pltpu.VMEM((NSUB, SUB), jnp.int32),
                       pltpu.VMEM((2, SUB, D), jnp.float32),
                       pltpu.VMEM((2, SUB, D), jnp.float32),
                       pltpu.SemaphoreType.DMA,
                       pltpu.SemaphoreType.DMA],
    )(_k4_body)
    return kern(ys, d0, d1)


def kernel(hidden_states, gate_weight, w1, w3, w2):
    x = hidden_states
    e0, e1, w0, w1t, r0, r1, h0, h1 = _run_router(x, gate_weight)
    block_expert, base0, base1 = _routing_tables(h0[:, 0, :], h1[:, 0, :])
    to16 = lambda a: a.reshape(T // SUB, SUB)
    xs, d0, d1, ws = _run_k2(x, to16(e0), to16(e1), to16(r0), to16(r1),
                             base0.reshape(NTB * E // 16, 16),
                             base1.reshape(NTB * E // 16, 16),
                             to16(w0), to16(w1t))
    ys = _run_k3(block_expert, xs, ws, w1, w3, w2)
    out = _run_k4(ys, d0, d1)
    return out
